# baseline (speedup 1.0000x reference)
import functools

import jax
import jax.numpy as jnp
from jax import lax
from jax.experimental import pallas as pl
from jax.experimental.pallas import tpu as pltpu
from jax.experimental.pallas import tpu_sc as plsc

NUM_MODULES = 11
HIDDEN = 256
N_NODES = 100000
B = 64
NSEG = B * NUM_MODULES
NSEG_PAD = 720

NW = 32
HALF = 50000
BLK = 10000
NBH = HALF // BLK
PER_W = 1600
HALF_PAD = NW * PER_W


def _stage1_body(x_ref, wt_ref, b2_ref, mod_ref, bat_ref,
                 raw_ref, proj_ref, seg_ref):
    x = x_ref[...]
    s = jax.lax.dot_general(
        x, wt_ref[...], (((1,), (0,)), ((), ())),
        preferred_element_type=jnp.float32)
    s = s + b2_ref[...]
    st = jnp.transpose(s)
    mod = mod_ref[0]
    row = jax.lax.broadcasted_iota(jnp.int32, (2 * NUM_MODULES, BLK), 0)
    raw_ref[0] = jnp.sum(jnp.where(row == mod, st, 0.0), axis=0,
                         keepdims=True)
    proj_ref[0] = jnp.sum(jnp.where(row == mod + NUM_MODULES, st, 0.0),
                          axis=0, keepdims=True)
    seg_ref[0] = bat_ref[0] * NUM_MODULES + mod


def _stage1(x, Wt, b2, mod3, bat3, boff):
    return pl.pallas_call(
        _stage1_body,
        grid=(NBH,),
        in_specs=[
            pl.BlockSpec((BLK, HIDDEN), lambda i: (i + boff, 0)),
            pl.BlockSpec((HIDDEN, 2 * NUM_MODULES), lambda i: (0, 0)),
            pl.BlockSpec((1, 2 * NUM_MODULES), lambda i: (0, 0)),
            pl.BlockSpec((1, 1, BLK), lambda i: (i, 0, 0)),
            pl.BlockSpec((1, 1, BLK), lambda i: (i, 0, 0)),
        ],
        out_specs=[
            pl.BlockSpec((1, 1, BLK), lambda i: (i, 0, 0)),
            pl.BlockSpec((1, 1, BLK), lambda i: (i, 0, 0)),
            pl.BlockSpec((1, 1, BLK), lambda i: (i, 0, 0)),
        ],
        out_shape=[
            jax.ShapeDtypeStruct((NBH, 1, BLK), jnp.float32),
            jax.ShapeDtypeStruct((NBH, 1, BLK), jnp.float32),
            jax.ShapeDtypeStruct((NBH, 1, BLK), jnp.int32),
        ],
    )(x, Wt, b2, mod3, bat3)


def _stage2(raw_p, proj_p, seg_p):
    mesh = plsc.VectorSubcoreMesh(core_axis_name="c", subcore_axis_name="s")

    @functools.partial(
        pl.kernel,
        mesh=mesh,
        compiler_params=pltpu.CompilerParams(needs_layout_passes=False),
        out_type=[
            jax.ShapeDtypeStruct((NW, 16), jnp.float32),
            jax.ShapeDtypeStruct((NW, NSEG_PAD), jnp.float32),
            jax.ShapeDtypeStruct((NW, NSEG_PAD), jnp.float32),
        ],
        scratch_types=[
            pltpu.VMEM((PER_W,), jnp.float32),
            pltpu.VMEM((PER_W,), jnp.float32),
            pltpu.VMEM((PER_W,), jnp.int32),
            pltpu.VMEM((16,), jnp.float32),
            pltpu.VMEM((NSEG_PAD,), jnp.float32),
            pltpu.VMEM((NSEG_PAD,), jnp.float32),
            pltpu.SemaphoreType.DMA,
            pltpu.SemaphoreType.DMA,
            pltpu.SemaphoreType.DMA,
        ],
    )
    def s2(raw_hbm, proj_hbm, seg_hbm, mo_hbm, do_hbm, no_hbm,
           raw_v, proj_v, seg_v, mw_v, d_v, n_v, sem1, sem2, sem3):
        wid = lax.axis_index("s") * 2 + lax.axis_index("c")
        base = wid * PER_W
        c1 = pltpu.async_copy(raw_hbm.at[pl.ds(base, PER_W)], raw_v, sem1)
        c2 = pltpu.async_copy(proj_hbm.at[pl.ds(base, PER_W)], proj_v, sem2)
        c3 = pltpu.async_copy(seg_hbm.at[pl.ds(base, PER_W)], seg_v, sem3)

        zero = jnp.zeros((16,), jnp.float32)
        for k in range(NSEG_PAD // 16):
            sl = pl.ds(k * 16, 16)
            d_v[sl] = zero
            n_v[sl] = zero

        c1.wait()
        UMAX = 4
        ninf = jnp.full((16,), -jnp.inf, jnp.float32)

        def max_body(k, accs):
            return tuple(
                jnp.maximum(accs[j], raw_v[pl.ds((UMAX * k + j) * 16, 16)])
                for j in range(UMAX))

        accs = lax.fori_loop(0, PER_W // 16 // UMAX, max_body,
                             (ninf,) * UMAX)
        macc = accs[0]
        for j in range(1, UMAX):
            macc = jnp.maximum(macc, accs[j])
        mw = lax.reduce_max(macc, (0,))
        mw_vec = lax.broadcast(mw, (16,))
        mw_v[...] = mw_vec

        c2.wait()
        c3.wait()
        U = 8

        def vec_body(k, carry):
            for j in range(U):
                sl = pl.ds((U * k + j) * 16, 16)
                sv = seg_v[sl]
                e = jnp.exp(raw_v[sl] - mw_vec)
                plsc.addupdate_scatter(d_v, [sv], e)
                plsc.addupdate_scatter(n_v, [sv], e * proj_v[sl])
            return carry

        lax.fori_loop(0, PER_W // 16 // U, vec_body, 0)

        pltpu.sync_copy(mw_v, mo_hbm.at[wid])
        pltpu.sync_copy(d_v, do_hbm.at[wid])
        pltpu.sync_copy(n_v, no_hbm.at[wid])

    return s2(raw_p, proj_p, seg_p)


def _stage3_body(m_ref, d_ref, n_ref, out_ref):
    m = m_ref[...][:, 0:1]
    gmax = jnp.max(m, axis=0, keepdims=True)
    scale = jnp.exp(m - gmax)
    denom = jnp.sum(d_ref[...] * scale, axis=0, keepdims=True)
    numer = jnp.sum(n_ref[...] * scale, axis=0, keepdims=True)
    out_ref[...] = numer / (denom + 1e-16)


def _stage3(mo, do, no):
    return pl.pallas_call(
        _stage3_body,
        out_shape=jax.ShapeDtypeStruct((1, NSEG_PAD), jnp.float32),
    )(mo, do, no)


def _pad_full(raw, proj, seg):
    pad = HALF_PAD - HALF
    raw_p = jnp.concatenate([raw.reshape(-1), jnp.zeros((pad,), jnp.float32)])
    proj_p = jnp.concatenate([proj.reshape(-1), jnp.zeros((pad,), jnp.float32)])
    seg_p = jnp.concatenate([seg.reshape(-1), jnp.full((pad,), NSEG, jnp.int32)])
    return raw_p, proj_p, seg_p


@jax.jit
def kernel(x, Wa, ba, Wp, bp, module_assign, batch):
    Wt = jnp.concatenate([Wa, Wp], axis=0).T
    b2 = jnp.concatenate([ba, bp]).reshape(1, 2 * NUM_MODULES)
    mod3 = module_assign.reshape(2, NBH, 1, BLK).astype(jnp.int32)
    bat3 = batch.reshape(2, NBH, 1, BLK).astype(jnp.int32)

    ra, pa, sa = _stage1(x, Wt, b2, mod3[0], bat3[0], 0)
    moa, doa, noa = _stage2(*_pad_full(ra, pa, sa))
    rb, pb, sb = _stage1(x, Wt, b2, mod3[1], bat3[1], NBH)
    mob, dob, nob = _stage2(*_pad_full(rb, pb, sb))

    mo = jnp.concatenate([moa, mob], axis=0)
    do = jnp.concatenate([doa, dob], axis=0)
    no = jnp.concatenate([noa, nob], axis=0)
    out = _stage3(mo, do, no)
    return out[0, :NSEG].reshape(B, NUM_MODULES)


# speedup vs baseline: 1.0956x; 1.0956x over previous
import functools

import jax
import jax.numpy as jnp
from jax import lax
from jax.experimental import pallas as pl
from jax.experimental.pallas import tpu as pltpu
from jax.experimental.pallas import tpu_sc as plsc

NUM_MODULES = 11
HIDDEN = 256
N_NODES = 100000
B = 64
NSEG = B * NUM_MODULES
NSEG_PAD = 720

NW = 32
BLK = 20000
NB = N_NODES // BLK
PER_W = 3200
N_PAD = NW * PER_W


def _stage1_body(x_ref, wt_ref, b2_ref, mod_ref, bat_ref,
                 raw_ref, proj_ref, seg_ref):
    x = x_ref[...]
    s = jax.lax.dot_general(
        x, wt_ref[...], (((1,), (0,)), ((), ())),
        preferred_element_type=jnp.float32)
    s = s + b2_ref[...]
    st = jnp.transpose(s)
    mod = mod_ref[0]
    row = jax.lax.broadcasted_iota(jnp.int32, (2 * NUM_MODULES, BLK), 0)
    raw_ref[0] = jnp.sum(jnp.where(row == mod, st, 0.0), axis=0,
                         keepdims=True)
    proj_ref[0] = jnp.sum(jnp.where(row == mod + NUM_MODULES, st, 0.0),
                          axis=0, keepdims=True)
    seg_ref[0] = bat_ref[0] * NUM_MODULES + mod


def _stage1(x, Wt, b2, mod3, bat3):
    return pl.pallas_call(
        _stage1_body,
        grid=(NB,),
        in_specs=[
            pl.BlockSpec((BLK, HIDDEN), lambda i: (i, 0)),
            pl.BlockSpec((HIDDEN, 2 * NUM_MODULES), lambda i: (0, 0)),
            pl.BlockSpec((1, 2 * NUM_MODULES), lambda i: (0, 0)),
            pl.BlockSpec((1, 1, BLK), lambda i: (i, 0, 0)),
            pl.BlockSpec((1, 1, BLK), lambda i: (i, 0, 0)),
        ],
        out_specs=[
            pl.BlockSpec((1, 1, BLK), lambda i: (i, 0, 0)),
            pl.BlockSpec((1, 1, BLK), lambda i: (i, 0, 0)),
            pl.BlockSpec((1, 1, BLK), lambda i: (i, 0, 0)),
        ],
        out_shape=[
            jax.ShapeDtypeStruct((NB, 1, BLK), jnp.float32),
            jax.ShapeDtypeStruct((NB, 1, BLK), jnp.float32),
            jax.ShapeDtypeStruct((NB, 1, BLK), jnp.int32),
        ],
    )(x, Wt, b2, mod3, bat3)


def _stage2(raw_p, proj_p, seg_p):
    mesh = plsc.VectorSubcoreMesh(core_axis_name="c", subcore_axis_name="s")

    @functools.partial(
        pl.kernel,
        mesh=mesh,
        compiler_params=pltpu.CompilerParams(needs_layout_passes=False),
        out_type=[
            jax.ShapeDtypeStruct((NW, 16), jnp.float32),
            jax.ShapeDtypeStruct((NW, NSEG_PAD), jnp.float32),
            jax.ShapeDtypeStruct((NW, NSEG_PAD), jnp.float32),
        ],
        scratch_types=[
            pltpu.VMEM((PER_W,), jnp.float32),
            pltpu.VMEM((PER_W,), jnp.float32),
            pltpu.VMEM((PER_W,), jnp.int32),
            pltpu.VMEM((16,), jnp.float32),
            pltpu.VMEM((NSEG_PAD,), jnp.float32),
            pltpu.VMEM((NSEG_PAD,), jnp.float32),
            pltpu.SemaphoreType.DMA,
            pltpu.SemaphoreType.DMA,
            pltpu.SemaphoreType.DMA,
        ],
    )
    def s2(raw_hbm, proj_hbm, seg_hbm, mo_hbm, do_hbm, no_hbm,
           raw_v, proj_v, seg_v, mw_v, d_v, n_v, sem1, sem2, sem3):
        wid = lax.axis_index("s") * 2 + lax.axis_index("c")
        base = wid * PER_W
        c1 = pltpu.async_copy(raw_hbm.at[pl.ds(base, PER_W)], raw_v, sem1)
        c2 = pltpu.async_copy(proj_hbm.at[pl.ds(base, PER_W)], proj_v, sem2)
        c3 = pltpu.async_copy(seg_hbm.at[pl.ds(base, PER_W)], seg_v, sem3)

        zero = jnp.zeros((16,), jnp.float32)
        for k in range(NSEG_PAD // 16):
            sl = pl.ds(k * 16, 16)
            d_v[sl] = zero
            n_v[sl] = zero

        c1.wait()
        UMAX = 4
        ninf = jnp.full((16,), -jnp.inf, jnp.float32)

        def max_body(k, accs):
            return tuple(
                jnp.maximum(accs[j], raw_v[pl.ds((UMAX * k + j) * 16, 16)])
                for j in range(UMAX))

        accs = lax.fori_loop(0, PER_W // 16 // UMAX, max_body,
                             (ninf,) * UMAX)
        macc = accs[0]
        for j in range(1, UMAX):
            macc = jnp.maximum(macc, accs[j])
        mw = lax.reduce_max(macc, (0,))
        mw_vec = lax.broadcast(mw, (16,))
        mw_v[...] = mw_vec

        c2.wait()
        c3.wait()
        U = 8

        def vec_body(k, carry):
            for j in range(U):
                sl = pl.ds((U * k + j) * 16, 16)
                sv = seg_v[sl]
                e = jnp.exp(raw_v[sl] - mw_vec)
                plsc.addupdate_scatter(d_v, [sv], e)
                plsc.addupdate_scatter(n_v, [sv], e * proj_v[sl])
            return carry

        lax.fori_loop(0, PER_W // 16 // U, vec_body, 0)

        pltpu.sync_copy(mw_v, mo_hbm.at[wid])
        pltpu.sync_copy(d_v, do_hbm.at[wid])
        pltpu.sync_copy(n_v, no_hbm.at[wid])

    return s2(raw_p, proj_p, seg_p)


def _stage3_body(m_ref, d_ref, n_ref, out_ref):
    m = m_ref[...][:, 0:1]
    gmax = jnp.max(m, axis=0, keepdims=True)
    scale = jnp.exp(m - gmax)
    denom = jnp.sum(d_ref[...] * scale, axis=0, keepdims=True)
    numer = jnp.sum(n_ref[...] * scale, axis=0, keepdims=True)
    out_ref[...] = numer / (denom + 1e-16)


def _stage3(mo, do, no):
    return pl.pallas_call(
        _stage3_body,
        out_shape=jax.ShapeDtypeStruct((1, NSEG_PAD), jnp.float32),
    )(mo, do, no)


def _pad_full(raw, proj, seg):
    pad = N_PAD - N_NODES
    raw_p = jnp.concatenate([raw.reshape(-1), jnp.zeros((pad,), jnp.float32)])
    proj_p = jnp.concatenate([proj.reshape(-1), jnp.zeros((pad,), jnp.float32)])
    seg_p = jnp.concatenate([seg.reshape(-1), jnp.full((pad,), NSEG, jnp.int32)])
    return raw_p, proj_p, seg_p


@jax.jit
def kernel(x, Wa, ba, Wp, bp, module_assign, batch):
    Wt = jnp.concatenate([Wa, Wp], axis=0).T
    b2 = jnp.concatenate([ba, bp]).reshape(1, 2 * NUM_MODULES)
    mod3 = module_assign.reshape(NB, 1, BLK).astype(jnp.int32)
    bat3 = batch.reshape(NB, 1, BLK).astype(jnp.int32)

    ra, pa, sa = _stage1(x, Wt, b2, mod3, bat3)
    mo, do, no = _stage2(*_pad_full(ra, pa, sa))
    out = _stage3(mo, do, no)
    return out[0, :NSEG].reshape(B, NUM_MODULES)


# speedup vs baseline: 1.1293x; 1.0307x over previous
import functools

import jax
import jax.numpy as jnp
from jax import lax
from jax.experimental import pallas as pl
from jax.experimental.pallas import tpu as pltpu
from jax.experimental.pallas import tpu_sc as plsc

NUM_MODULES = 11
HIDDEN = 256
N_NODES = 100000
B = 64
NSEG = B * NUM_MODULES
NSEG_PAD = 720

NW = 32
BLK = 10000
NB = N_NODES // BLK
PER_W = 3200
N_PAD = NW * PER_W


def _stage1_body(x_ref, wt_ref, b2_ref, mod_ref, bat_ref,
                 raw_ref, proj_ref, seg_ref):
    x = x_ref[...]
    s = jax.lax.dot_general(
        x, wt_ref[...], (((1,), (0,)), ((), ())),
        preferred_element_type=jnp.float32)
    s = s + b2_ref[...]
    st = jnp.transpose(s)
    mod = mod_ref[0]
    row = jax.lax.broadcasted_iota(jnp.int32, (2 * NUM_MODULES, BLK), 0)
    raw_ref[0] = jnp.sum(jnp.where(row == mod, st, 0.0), axis=0,
                         keepdims=True)
    proj_ref[0] = jnp.sum(jnp.where(row == mod + NUM_MODULES, st, 0.0),
                          axis=0, keepdims=True)
    seg_ref[0] = bat_ref[0] * NUM_MODULES + mod


def _stage1(x, Wt, b2, mod3, bat3):
    return pl.pallas_call(
        _stage1_body,
        grid=(NB,),
        in_specs=[
            pl.BlockSpec((BLK, HIDDEN), lambda i: (i, 0)),
            pl.BlockSpec((HIDDEN, 2 * NUM_MODULES), lambda i: (0, 0)),
            pl.BlockSpec((1, 2 * NUM_MODULES), lambda i: (0, 0)),
            pl.BlockSpec((1, 1, BLK), lambda i: (i, 0, 0)),
            pl.BlockSpec((1, 1, BLK), lambda i: (i, 0, 0)),
        ],
        out_specs=[
            pl.BlockSpec((1, 1, BLK), lambda i: (i, 0, 0)),
            pl.BlockSpec((1, 1, BLK), lambda i: (i, 0, 0)),
            pl.BlockSpec((1, 1, BLK), lambda i: (i, 0, 0)),
        ],
        out_shape=[
            jax.ShapeDtypeStruct((NB, 1, BLK), jnp.float32),
            jax.ShapeDtypeStruct((NB, 1, BLK), jnp.float32),
            jax.ShapeDtypeStruct((NB, 1, BLK), jnp.int32),
        ],
    )(x, Wt, b2, mod3, bat3)


def _stage2(raw_p, proj_p, seg_p):
    mesh = plsc.VectorSubcoreMesh(core_axis_name="c", subcore_axis_name="s")

    @functools.partial(
        pl.kernel,
        mesh=mesh,
        compiler_params=pltpu.CompilerParams(needs_layout_passes=False),
        out_type=[
            jax.ShapeDtypeStruct((NW, NSEG_PAD), jnp.float32),
            jax.ShapeDtypeStruct((NW, NSEG_PAD), jnp.float32),
        ],
        scratch_types=[
            pltpu.VMEM((PER_W,), jnp.float32),
            pltpu.VMEM((PER_W,), jnp.float32),
            pltpu.VMEM((PER_W,), jnp.int32),
            pltpu.VMEM((16,), jnp.float32),
            pltpu.VMEM((NSEG_PAD,), jnp.float32),
            pltpu.VMEM((NSEG_PAD,), jnp.float32),
            pltpu.SemaphoreType.DMA,
            pltpu.SemaphoreType.DMA,
            pltpu.SemaphoreType.DMA,
        ],
    )
    def s2(raw_hbm, proj_hbm, seg_hbm, do_hbm, no_hbm,
           raw_v, proj_v, seg_v, mw_v, d_v, n_v, sem1, sem2, sem3):
        wid = lax.axis_index("s") * 2 + lax.axis_index("c")
        base = wid * PER_W
        c1 = pltpu.async_copy(raw_hbm.at[pl.ds(base, PER_W)], raw_v, sem1)
        c2 = pltpu.async_copy(proj_hbm.at[pl.ds(base, PER_W)], proj_v, sem2)
        c3 = pltpu.async_copy(seg_hbm.at[pl.ds(base, PER_W)], seg_v, sem3)

        zero = jnp.zeros((16,), jnp.float32)
        for k in range(NSEG_PAD // 16):
            sl = pl.ds(k * 16, 16)
            d_v[sl] = zero
            n_v[sl] = zero

        c1.wait()
        UMAX = 4
        ninf = jnp.full((16,), -jnp.inf, jnp.float32)

        def max_body(k, accs):
            return tuple(
                jnp.maximum(accs[j], raw_v[pl.ds((UMAX * k + j) * 16, 16)])
                for j in range(UMAX))

        accs = lax.fori_loop(0, PER_W // 16 // UMAX, max_body,
                             (ninf,) * UMAX)
        macc = accs[0]
        for j in range(1, UMAX):
            macc = jnp.maximum(macc, accs[j])
        mw = lax.reduce_max(macc, (0,))
        mw_vec = lax.broadcast(mw, (16,))
        mw_v[...] = mw_vec

        c2.wait()
        c3.wait()
        U = 8

        def vec_body(k, carry):
            for j in range(U):
                sl = pl.ds((U * k + j) * 16, 16)
                sv = seg_v[sl]
                e = jnp.exp(raw_v[sl] - mw_vec)
                plsc.addupdate_scatter(d_v, [sv], e)
                plsc.addupdate_scatter(n_v, [sv], e * proj_v[sl])
            return carry

        lax.fori_loop(0, PER_W // 16 // U, vec_body, 0)

        d_v[pl.ds(NSEG, 16)] = mw_vec
        pltpu.sync_copy(d_v, do_hbm.at[wid])
        pltpu.sync_copy(n_v, no_hbm.at[wid])

    return s2(raw_p, proj_p, seg_p)


def _stage3_body(d_ref, n_ref, out_ref):
    m = d_ref[...][:, NSEG:NSEG + 1]
    gmax = jnp.max(m, axis=0, keepdims=True)
    scale = jnp.exp(m - gmax)
    denom = jnp.sum(d_ref[...] * scale, axis=0, keepdims=True)
    numer = jnp.sum(n_ref[...] * scale, axis=0, keepdims=True)
    out_ref[...] = numer / (denom + 1e-16)


def _stage3(do, no):
    return pl.pallas_call(
        _stage3_body,
        out_shape=jax.ShapeDtypeStruct((1, NSEG_PAD), jnp.float32),
    )(do, no)


def _pad_full(raw, proj, seg):
    pad = N_PAD - N_NODES
    raw_p = jnp.concatenate([raw.reshape(-1), jnp.zeros((pad,), jnp.float32)])
    proj_p = jnp.concatenate([proj.reshape(-1), jnp.zeros((pad,), jnp.float32)])
    seg_p = jnp.concatenate([seg.reshape(-1), jnp.full((pad,), NSEG, jnp.int32)])
    return raw_p, proj_p, seg_p


@jax.jit
def kernel(x, Wa, ba, Wp, bp, module_assign, batch):
    Wt = jnp.concatenate([Wa, Wp], axis=0).T
    b2 = jnp.concatenate([ba, bp]).reshape(1, 2 * NUM_MODULES)
    mod3 = module_assign.reshape(NB, 1, BLK).astype(jnp.int32)
    bat3 = batch.reshape(NB, 1, BLK).astype(jnp.int32)

    ra, pa, sa = _stage1(x, Wt, b2, mod3, bat3)
    do, no = _stage2(*_pad_full(ra, pa, sa))
    out = _stage3(do, no)
    return out[0, :NSEG].reshape(B, NUM_MODULES)
